# Initial kernel scaffold; baseline (speedup 1.0000x reference)
import jax
import jax.numpy as jnp
from jax.experimental import pallas as pl


def kernel(x, W_router, gate_w, up_w, down_w, shared_gate, shared_up, shared_down):
    raise NotImplementedError("write your pallas kernel here")


# speedup vs baseline: 2.2603x; 2.2603x over previous
import functools

import jax
import jax.numpy as jnp
from jax import lax
from jax.experimental import pallas as pl

B, S, H, FF, E = 1, 2048, 1024, 512, 16
TOP_K, N_GROUP, TOPK_GROUP = 2, 8, 4
SCALING = 2.5
T = B * S


def _router_kernel(x_ref, wr_ref, w_ref):
    x = x_ref[...]
    wr = wr_ref[...]
    logits = lax.dot_general(x, wr, (((1,), (1,)), ((), ())),
                             preferred_element_type=jnp.float32)
    scores = jax.nn.sigmoid(logits)

    iota = lax.broadcasted_iota(jnp.int32, (T, E), 1)
    r16 = lax.broadcasted_iota(jnp.int32, (E, E), 0)
    c16 = lax.broadcasted_iota(jnp.int32, (E, E), 1)
    partner_idx = r16 + 1 - 2 * (r16 % 2)
    P = (partner_idx == c16).astype(jnp.float32)
    partner = lax.dot_general(scores, P, (((1,), (0,)), ((), ())),
                              preferred_element_type=jnp.float32)
    gs = jnp.maximum(scores, partner)
    giota = iota // 2

    group_mask = jnp.zeros((T, E), dtype=jnp.bool_)
    g = gs
    for _ in range(TOPK_GROUP):
        m = jnp.max(g, axis=-1, keepdims=True)
        cand = jnp.where(g == m, giota, N_GROUP)
        sel = jnp.min(cand, axis=-1, keepdims=True)
        hit = giota == sel
        group_mask = group_mask | hit
        g = jnp.where(hit, -1.0, g)

    routed = jnp.where(group_mask, scores, 0.0)

    r = routed
    wsum = jnp.zeros((T, 1), dtype=jnp.float32)
    picks = []
    for _ in range(TOP_K):
        m = jnp.max(r, axis=-1, keepdims=True)
        cand = jnp.where(r == m, iota, E)
        sel = jnp.min(cand, axis=-1, keepdims=True)
        hit = iota == sel
        picks.append((hit, m))
        wsum = wsum + m
        r = jnp.where(hit, -1.0, r)

    scale = SCALING / jnp.maximum(wsum, 1e-9)
    w_full = jnp.zeros((T, E), dtype=jnp.float32)
    for hit, m in picks:
        w_full = w_full + jnp.where(hit, m * scale, 0.0)
    w_ref[...] = w_full


def _dense_moe_kernel(x_ref, g_ref, u_ref, d_ref, w_ref, out_ref):
    e = pl.program_id(0)

    @pl.when(e == 0)
    def _init():
        out_ref[...] = jnp.zeros_like(out_ref)

    x = x_ref[...]
    gw = g_ref[0]
    uw = u_ref[0]
    dw = d_ref[0]
    w = w_ref[0, 0]
    a = lax.dot_general(x, gw, (((1,), (1,)), ((), ())),
                        preferred_element_type=jnp.float32)
    b = lax.dot_general(x, uw, (((1,), (1,)), ((), ())),
                        preferred_element_type=jnp.float32)
    h = a * jax.nn.sigmoid(a) * b
    o = lax.dot_general(h, dw, (((1,), (1,)), ((), ())),
                        preferred_element_type=jnp.float32)
    out_ref[...] += o * w[:, None]


def kernel(x, W_router, gate_w, up_w, down_w, shared_gate, shared_up, shared_down):
    b, s, h = x.shape
    tokens = x.reshape(T, H)

    w_full = pl.pallas_call(
        _router_kernel,
        out_shape=jax.ShapeDtypeStruct((T, E), jnp.float32),
    )(tokens, W_router)

    gall = jnp.concatenate([gate_w, shared_gate[None]], axis=0)
    uall = jnp.concatenate([up_w, shared_up[None]], axis=0)
    dall = jnp.concatenate([down_w, shared_down[None]], axis=0)
    w_ext = jnp.concatenate([w_full.T, jnp.ones((1, T), jnp.float32)],
                            axis=0).reshape(E + 1, 1, T)

    out = pl.pallas_call(
        _dense_moe_kernel,
        grid=(E + 1,),
        in_specs=[
            pl.BlockSpec((T, H), lambda e: (0, 0)),
            pl.BlockSpec((1, FF, H), lambda e: (e, 0, 0)),
            pl.BlockSpec((1, FF, H), lambda e: (e, 0, 0)),
            pl.BlockSpec((1, H, FF), lambda e: (e, 0, 0)),
            pl.BlockSpec((1, 1, T), lambda e: (e, 0, 0)),
        ],
        out_specs=pl.BlockSpec((T, H), lambda e: (0, 0)),
        out_shape=jax.ShapeDtypeStruct((T, H), jnp.float32),
    )(tokens, gall, uall, dall, w_ext)

    return out.reshape(b, s, h)
